# baseline (speedup 1.0000x reference)
import functools

import jax
import jax.numpy as jnp
from jax import lax
from jax.experimental import pallas as pl
from jax.experimental.pallas import tpu as pltpu
from jax.experimental.pallas import tpu_sc as plsc

_L = 128
_QB = 32
_NSUB = 16
_NCORE = 2
_NW = _NSUB * _NCORE
_BL = 1000


def _dot_t(a, w):
    return lax.dot_general(a, w, (((1,), (1,)), ((), ())),
                           preferred_element_type=jnp.float32)


def _tc_entry(x, Wl, Wr, b2d):
    N, D = x.shape

    def body(x_ref, wl_ref, wr_ref, b_ref, z_ref, r_ref):
        xb = x_ref[...]
        z_ref[...] = _dot_t(xb, wl_ref[...])
        r_ref[...] = _dot_t(xb, wr_ref[...]) + b_ref[...]

    return pl.pallas_call(
        body,
        grid=(N // _BL,),
        in_specs=[
            pl.BlockSpec((_BL, D), lambda i: (i, 0)),
            pl.BlockSpec((D, D), lambda i: (0, 0)),
            pl.BlockSpec((D, D), lambda i: (0, 0)),
            pl.BlockSpec((1, D), lambda i: (0, 0)),
        ],
        out_specs=[pl.BlockSpec((_BL, D), lambda i: (i, 0)),
                   pl.BlockSpec((_BL, D), lambda i: (i, 0))],
        out_shape=[jax.ShapeDtypeStruct((N, D), jnp.float32)] * 2,
    )(x, Wl, Wr, b2d)


def _tc_mid(s_part, cnt_t, r_prev, Wl, Wr, b2d):
    _, N, D = s_part.shape

    def body(s_ref, c_ref, r_ref, wl_ref, wr_ref, b_ref, z_ref, r2_ref):
        cb = c_ref[...]
        rc = 1.0 / jnp.maximum(cb[:, 0] + cb[:, 1], 1.0)
        h = jnp.maximum((s_ref[0] + s_ref[1]) * rc[:, None] + r_ref[...], 0.0)
        z_ref[...] = _dot_t(h, wl_ref[...])
        r2_ref[...] = _dot_t(h, wr_ref[...]) + b_ref[...]

    return pl.pallas_call(
        body,
        grid=(N // _BL,),
        in_specs=[
            pl.BlockSpec((2, _BL, D), lambda i: (0, i, 0)),
            pl.BlockSpec((_BL, 2), lambda i: (i, 0)),
            pl.BlockSpec((_BL, D), lambda i: (i, 0)),
            pl.BlockSpec((D, D), lambda i: (0, 0)),
            pl.BlockSpec((D, D), lambda i: (0, 0)),
            pl.BlockSpec((1, D), lambda i: (0, 0)),
        ],
        out_specs=[pl.BlockSpec((_BL, D), lambda i: (i, 0)),
                   pl.BlockSpec((_BL, D), lambda i: (i, 0))],
        out_shape=[jax.ShapeDtypeStruct((N, D), jnp.float32)] * 2,
    )(s_part, cnt_t, r_prev, Wl, Wr, b2d)


def _tc_final(s_part, cnt_t, r_prev):
    _, N, D = s_part.shape

    def body(s_ref, c_ref, r_ref, o_ref):
        cb = c_ref[...]
        rc = 1.0 / jnp.maximum(cb[:, 0] + cb[:, 1], 1.0)
        h = (s_ref[0] + s_ref[1]) * rc[:, None] + r_ref[...]
        m = jnp.max(h, axis=1, keepdims=True)
        lse = jnp.log(jnp.sum(jnp.exp(h - m), axis=1, keepdims=True))
        o_ref[...] = h - m - lse

    return pl.pallas_call(
        body,
        grid=(N // _BL,),
        in_specs=[
            pl.BlockSpec((2, _BL, D), lambda i: (0, i, 0)),
            pl.BlockSpec((_BL, 2), lambda i: (i, 0)),
            pl.BlockSpec((_BL, D), lambda i: (i, 0)),
        ],
        out_specs=pl.BlockSpec((_BL, D), lambda i: (i, 0)),
        out_shape=jax.ShapeDtypeStruct((N, D), jnp.float32),
    )(s_part, cnt_t, r_prev)


def _sc_segment_sum(z, src2d, dst2d, zeros_rows, zeros_flat, ones_vec,
                    with_cnt: bool):
    N, D = z.shape
    EC = src2d.shape[0]
    q = EC // _NW
    assert EC % _NW == 0 and q % _QB == 0
    RS = (N // _NSUB) // 8 * 8
    RREM = N - RS * _NSUB
    ZREM = N + _NSUB - RS * _NSUB
    ZC = zeros_rows.shape[0]
    assert RS % ZC == 0 and ZC % 8 == 0 and ZREM <= ZC
    CPAD = zeros_flat.shape[0]
    NPAD = CPAD * _NSUB

    mesh = plsc.VectorSubcoreMesh(core_axis_name="c", subcore_axis_name="s")

    out_type = [jax.ShapeDtypeStruct((_NCORE, N, D), jnp.float32)]
    scratch = [
        pltpu.VMEM_SHARED((N + _NSUB, D), jnp.float32),
        pltpu.VMEM((_QB, _L), jnp.int32),
        pltpu.VMEM((_QB, _L), jnp.int32),
        pltpu.VMEM((_L, D), jnp.float32),
        pltpu.VMEM((_L, D), jnp.float32),
        pltpu.VMEM((ZC, D), jnp.float32),
        pltpu.SemaphoreType.DMA,
        pltpu.SemaphoreType.DMA,
    ]
    if with_cnt:
        out_type.append(jax.ShapeDtypeStruct((_NCORE * NPAD,), jnp.float32))
        scratch += [
            pltpu.VMEM_SHARED((NPAD,), jnp.float32),
            pltpu.VMEM((CPAD,), jnp.float32),
            pltpu.VMEM((_L,), jnp.float32),
        ]

    def body(z_h, src_h, dst_h, zr_h, zf_h, on_h, s_out, *rest):
        if with_cnt:
            (cnt_out, acc_sh, src_all, dst_all, rows0, rows1, stage_v,
             sem0, sem1, cnt_sh, cnt_stage, ones_v) = rest
        else:
            acc_sh, src_all, dst_all, rows0, rows1, stage_v, sem0, sem1 = rest
        c = lax.axis_index("c")
        s = lax.axis_index("s")
        w = s * _NCORE + c

        pltpu.sync_copy(zr_h, stage_v)
        for k in range(RS // ZC):
            pltpu.sync_copy(stage_v, acc_sh.at[pl.ds(s * RS + k * ZC, ZC), :])

        @pl.when(s == 0)
        def _():
            pltpu.sync_copy(stage_v.at[pl.ds(0, ZREM), :],
                            acc_sh.at[pl.ds(RS * _NSUB, ZREM), :])
        if with_cnt:
            pltpu.sync_copy(zf_h, cnt_stage)
            pltpu.sync_copy(cnt_stage, cnt_sh.at[pl.ds(s * CPAD, CPAD)])
            pltpu.sync_copy(on_h, ones_v)

        plsc.subcore_barrier()

        def block(b, carry):
            base = w * q + b * _QB
            pltpu.sync_copy(src_h.at[pl.ds(base, _QB), :], src_all)
            pltpu.sync_copy(dst_h.at[pl.ds(base, _QB), :], dst_all)

            for j in range(_QB):
                rows = rows0 if j % 2 == 0 else rows1
                pltpu.sync_copy(z_h.at[src_all.at[j]], rows)
                pltpu.sync_copy(rows, acc_sh.at[dst_all.at[j]], add=True)
                if with_cnt:
                    pltpu.sync_copy(ones_v, cnt_sh.at[dst_all.at[j]],
                                    add=True)
            return carry

        lax.fori_loop(0, q // _QB, block, 0)

        plsc.subcore_barrier()

        for k in range(RS // ZC):
            pltpu.sync_copy(acc_sh.at[pl.ds(s * RS + k * ZC, ZC), :], stage_v)
            pltpu.sync_copy(stage_v, s_out.at[c, pl.ds(s * RS + k * ZC, ZC), :])

        @pl.when(s == 0)
        def _():
            pltpu.sync_copy(acc_sh.at[pl.ds(RS * _NSUB, RREM), :],
                            stage_v.at[pl.ds(0, RREM), :])
            pltpu.sync_copy(stage_v.at[pl.ds(0, RREM), :],
                            s_out.at[c, pl.ds(RS * _NSUB, RREM), :])
        if with_cnt:
            pltpu.sync_copy(cnt_sh.at[pl.ds(s * CPAD, CPAD)], cnt_stage)
            pltpu.sync_copy(cnt_stage,
                            cnt_out.at[pl.ds(c * NPAD + s * CPAD, CPAD)])

    return pl.kernel(body, out_type=out_type, mesh=mesh,
                     scratch_types=scratch)(
        z, src2d, dst2d, zeros_rows, zeros_flat, ones_vec)


def kernel(x, edge_index, W1l, b1, W1r, W2l, b2, W2r, W3l, b3, W3r):
    N, D = x.shape
    E = edge_index.shape[1]
    assert N % _NSUB == 0

    EC = -(-E // _L)
    q = -(-EC // _NW)
    q = -(-q // _QB) * _QB
    EP = q * _NW * _L
    pad_dst = N + jnp.arange(EP - E, dtype=jnp.int32) % _NSUB
    src_p = jnp.concatenate(
        [edge_index[0], jnp.zeros((EP - E,), jnp.int32)])
    dst_p = jnp.concatenate([edge_index[1], pad_dst])
    src2d = src_p.reshape(EP // _L, _L)
    dst2d = dst_p.reshape(EP // _L, _L)

    RS = (N // _NSUB) // 8 * 8
    ZC = next((c for c in (48, 24, 16, 8) if RS % c == 0), RS)
    CPAD = ((N // _NSUB + 7) // 8) * 8
    NPAD = CPAD * _NSUB
    zeros_rows = jnp.zeros((ZC, D), jnp.float32)
    zeros_flat = jnp.zeros((CPAD,), jnp.float32)
    ones_vec = jnp.ones((_L,), jnp.float32)

    b1d = b1.reshape(1, D)
    b2d = b2.reshape(1, D)
    b3d = b3.reshape(1, D)

    z1, r1 = _tc_entry(x, W1l, W1r, b1d)
    s1, cnt_pad = _sc_segment_sum(z1, src2d, dst2d, zeros_rows, zeros_flat,
                                  ones_vec, with_cnt=True)
    cnt_t = cnt_pad.reshape(_NCORE, NPAD)[:, :N].T

    z2, r2 = _tc_mid(s1, cnt_t, r1, W2l, W2r, b2d)
    s2 = _sc_segment_sum(z2, src2d, dst2d, zeros_rows, zeros_flat,
                         ones_vec, with_cnt=False)[0]

    z3, r3 = _tc_mid(s2, cnt_t, r2, W3l, W3r, b3d)
    s3 = _sc_segment_sum(z3, src2d, dst2d, zeros_rows, zeros_flat,
                         ones_vec, with_cnt=False)[0]

    return _tc_final(s3, cnt_t, r3)


# speedup vs baseline: 5.1839x; 5.1839x over previous
import functools

import jax
import jax.numpy as jnp
from jax import lax
from jax.experimental import pallas as pl
from jax.experimental.pallas import tpu as pltpu
from jax.experimental.pallas import tpu_sc as plsc

_L = 128
_QB = 8
_NSUB = 16
_NCORE = 2
_NW = _NSUB * _NCORE
_BL = 1000


def _dot_t(a, w):
    return lax.dot_general(a, w, (((1,), (1,)), ((), ())),
                           preferred_element_type=jnp.float32)


def _tc_entry(x, Wl, Wr, b2d):
    N, D = x.shape

    def body(x_ref, wl_ref, wr_ref, b_ref, z_ref, r_ref):
        xb = x_ref[...]
        z_ref[...] = _dot_t(xb, wl_ref[...])
        r_ref[...] = _dot_t(xb, wr_ref[...]) + b_ref[...]

    return pl.pallas_call(
        body,
        grid=(N // _BL,),
        in_specs=[
            pl.BlockSpec((_BL, D), lambda i: (i, 0)),
            pl.BlockSpec((D, D), lambda i: (0, 0)),
            pl.BlockSpec((D, D), lambda i: (0, 0)),
            pl.BlockSpec((1, D), lambda i: (0, 0)),
        ],
        out_specs=[pl.BlockSpec((_BL, D), lambda i: (i, 0)),
                   pl.BlockSpec((_BL, D), lambda i: (i, 0))],
        out_shape=[jax.ShapeDtypeStruct((N, D), jnp.float32)] * 2,
    )(x, Wl, Wr, b2d)


def _tc_mid(s_part, cnt_t, r_prev, Wl, Wr, b2d):
    _, N, D = s_part.shape

    def body(s_ref, c_ref, r_ref, wl_ref, wr_ref, b_ref, z_ref, r2_ref):
        cb = c_ref[...]
        rc = 1.0 / jnp.maximum(cb[:, 0] + cb[:, 1], 1.0)
        h = jnp.maximum((s_ref[0] + s_ref[1]) * rc[:, None] + r_ref[...], 0.0)
        z_ref[...] = _dot_t(h, wl_ref[...])
        r2_ref[...] = _dot_t(h, wr_ref[...]) + b_ref[...]

    return pl.pallas_call(
        body,
        grid=(N // _BL,),
        in_specs=[
            pl.BlockSpec((2, _BL, D), lambda i: (0, i, 0)),
            pl.BlockSpec((_BL, 2), lambda i: (i, 0)),
            pl.BlockSpec((_BL, D), lambda i: (i, 0)),
            pl.BlockSpec((D, D), lambda i: (0, 0)),
            pl.BlockSpec((D, D), lambda i: (0, 0)),
            pl.BlockSpec((1, D), lambda i: (0, 0)),
        ],
        out_specs=[pl.BlockSpec((_BL, D), lambda i: (i, 0)),
                   pl.BlockSpec((_BL, D), lambda i: (i, 0))],
        out_shape=[jax.ShapeDtypeStruct((N, D), jnp.float32)] * 2,
    )(s_part, cnt_t, r_prev, Wl, Wr, b2d)


def _tc_final(s_part, cnt_t, r_prev):
    _, N, D = s_part.shape

    def body(s_ref, c_ref, r_ref, o_ref):
        cb = c_ref[...]
        rc = 1.0 / jnp.maximum(cb[:, 0] + cb[:, 1], 1.0)
        h = (s_ref[0] + s_ref[1]) * rc[:, None] + r_ref[...]
        m = jnp.max(h, axis=1, keepdims=True)
        lse = jnp.log(jnp.sum(jnp.exp(h - m), axis=1, keepdims=True))
        o_ref[...] = h - m - lse

    return pl.pallas_call(
        body,
        grid=(N // _BL,),
        in_specs=[
            pl.BlockSpec((2, _BL, D), lambda i: (0, i, 0)),
            pl.BlockSpec((_BL, 2), lambda i: (i, 0)),
            pl.BlockSpec((_BL, D), lambda i: (i, 0)),
        ],
        out_specs=pl.BlockSpec((_BL, D), lambda i: (i, 0)),
        out_shape=jax.ShapeDtypeStruct((N, D), jnp.float32),
    )(s_part, cnt_t, r_prev)


def _sc_segment_sum(z, src2d, dst2d, zeros_rows, zeros_flat, ones_vec,
                    with_cnt: bool):
    N, D = z.shape
    EC = src2d.shape[0]
    q = EC // _NW
    assert EC % _NW == 0 and q % _QB == 0
    RS = (N // _NSUB) // 8 * 8
    RREM = N - RS * _NSUB
    ZREM = N + _NSUB - RS * _NSUB
    ZC = zeros_rows.shape[0]
    assert RS % ZC == 0 and ZC % 8 == 0 and ZREM <= ZC
    CPAD = zeros_flat.shape[0]
    NPAD = CPAD * _NSUB

    mesh = plsc.VectorSubcoreMesh(core_axis_name="c", subcore_axis_name="s")

    out_type = [jax.ShapeDtypeStruct((_NCORE, N, D), jnp.float32)]
    scratch = [
        pltpu.VMEM_SHARED((N + _NSUB, D), jnp.float32),
        pltpu.VMEM((_QB, _L), jnp.int32),
        pltpu.VMEM((_QB, _L), jnp.int32),
        pltpu.VMEM((_L, D), jnp.float32),
        pltpu.VMEM((_L, D), jnp.float32),
        pltpu.VMEM((ZC, D), jnp.float32),
        pltpu.SemaphoreType.DMA,
        pltpu.SemaphoreType.DMA,
    ]
    if with_cnt:
        out_type.append(jax.ShapeDtypeStruct((_NCORE * NPAD,), jnp.float32))
        scratch += [
            pltpu.VMEM_SHARED((NPAD,), jnp.float32),
            pltpu.VMEM((CPAD,), jnp.float32),
            pltpu.VMEM((_L,), jnp.float32),
        ]

    def body(z_h, src_h, dst_h, zr_h, zf_h, on_h, s_out, *rest):
        if with_cnt:
            (cnt_out, acc_sh, src_all, dst_all, rows0, rows1, stage_v,
             sem0, sem1, cnt_sh, cnt_stage, ones_v) = rest
        else:
            acc_sh, src_all, dst_all, rows0, rows1, stage_v, sem0, sem1 = rest
        c = lax.axis_index("c")
        s = lax.axis_index("s")
        w = s * _NCORE + c

        pltpu.sync_copy(zr_h, stage_v)
        for k in range(RS // ZC):
            pltpu.sync_copy(stage_v, acc_sh.at[pl.ds(s * RS + k * ZC, ZC), :])

        @pl.when(s == 0)
        def _():
            pltpu.sync_copy(stage_v.at[pl.ds(0, ZREM), :],
                            acc_sh.at[pl.ds(RS * _NSUB, ZREM), :])
        if with_cnt:
            pltpu.sync_copy(zf_h, cnt_stage)
            pltpu.sync_copy(cnt_stage, cnt_sh.at[pl.ds(s * CPAD, CPAD)])
            pltpu.sync_copy(on_h, ones_v)

        plsc.subcore_barrier()

        def block(b, carry):
            base = w * q + b * _QB
            pltpu.sync_copy(src_h.at[pl.ds(base, _QB), :], src_all)
            pltpu.sync_copy(dst_h.at[pl.ds(base, _QB), :], dst_all)

            for j in range(_QB):
                rows = rows0 if j % 2 == 0 else rows1
                pltpu.sync_copy(z_h.at[src_all.at[j]], rows)
                pltpu.sync_copy(rows, acc_sh.at[dst_all.at[j]], add=True)
                if with_cnt:
                    pltpu.sync_copy(ones_v, cnt_sh.at[dst_all.at[j]],
                                    add=True)
            return carry

        lax.fori_loop(0, q // _QB, block, 0)

        plsc.subcore_barrier()

        for k in range(RS // ZC):
            pltpu.sync_copy(acc_sh.at[pl.ds(s * RS + k * ZC, ZC), :], stage_v)
            pltpu.sync_copy(stage_v, s_out.at[c, pl.ds(s * RS + k * ZC, ZC), :])

        @pl.when(s == 0)
        def _():
            pltpu.sync_copy(acc_sh.at[pl.ds(RS * _NSUB, RREM), :],
                            stage_v.at[pl.ds(0, RREM), :])
            pltpu.sync_copy(stage_v.at[pl.ds(0, RREM), :],
                            s_out.at[c, pl.ds(RS * _NSUB, RREM), :])
        if with_cnt:
            pltpu.sync_copy(cnt_sh.at[pl.ds(s * CPAD, CPAD)], cnt_stage)
            pltpu.sync_copy(cnt_stage,
                            cnt_out.at[pl.ds(c * NPAD + s * CPAD, CPAD)])

    return pl.kernel(body, out_type=out_type, mesh=mesh,
                     scratch_types=scratch)(
        z, src2d, dst2d, zeros_rows, zeros_flat, ones_vec)


def kernel(x, edge_index, W1l, b1, W1r, W2l, b2, W2r, W3l, b3, W3r):
    N, D = x.shape
    E = edge_index.shape[1]
    assert N % _NSUB == 0

    EC = -(-E // _L)
    q = -(-EC // _NW)
    q = -(-q // _QB) * _QB
    EP = q * _NW * _L
    pad_dst = N + jnp.arange(EP - E, dtype=jnp.int32) % _NSUB
    src_p = jnp.concatenate(
        [edge_index[0], jnp.zeros((EP - E,), jnp.int32)])
    dst_p = jnp.concatenate([edge_index[1], pad_dst])
    src2d = src_p.reshape(EP // _L, _L)
    dst2d = dst_p.reshape(EP // _L, _L)

    RS = (N // _NSUB) // 8 * 8
    ZC = next((c for c in (48, 24, 16, 8) if RS % c == 0), RS)
    CPAD = ((N // _NSUB + 7) // 8) * 8
    NPAD = CPAD * _NSUB
    zeros_rows = jnp.zeros((ZC, D), jnp.float32)
    zeros_flat = jnp.zeros((CPAD,), jnp.float32)
    ones_vec = jnp.ones((_L,), jnp.float32)

    b1d = b1.reshape(1, D)
    b2d = b2.reshape(1, D)
    b3d = b3.reshape(1, D)

    z1, r1 = _tc_entry(x, W1l, W1r, b1d)
    s1, cnt_pad = _sc_segment_sum(z1, src2d, dst2d, zeros_rows, zeros_flat,
                                  ones_vec, with_cnt=True)
    cnt_t = cnt_pad.reshape(_NCORE, NPAD)[:, :N].T

    z2, r2 = _tc_mid(s1, cnt_t, r1, W2l, W2r, b2d)
    s2 = _sc_segment_sum(z2, src2d, dst2d, zeros_rows, zeros_flat,
                         ones_vec, with_cnt=False)[0]

    z3, r3 = _tc_mid(s2, cnt_t, r2, W3l, W3r, b3d)
    s3 = _sc_segment_sum(z3, src2d, dst2d, zeros_rows, zeros_flat,
                         ones_vec, with_cnt=False)[0]

    return _tc_final(s3, cnt_t, r3)


# speedup vs baseline: 12.9784x; 2.5036x over previous
import functools

import jax
import jax.numpy as jnp
import numpy as np
from jax import lax
from jax.experimental import pallas as pl
from jax.experimental.pallas import tpu as pltpu
from jax.experimental.pallas import tpu_sc as plsc

_L = 128
_QB = 8
_NSUB = 16
_NCORE = 2
_NW = _NSUB * _NCORE
_BL = 1000


def _dot_t(a, w):
    return lax.dot_general(a, w, (((1,), (1,)), ((), ())),
                           preferred_element_type=jnp.float32)


def _tc_entry(x, Wl, Wr, b2d):
    N, D = x.shape

    def body(x_ref, wl_ref, wr_ref, b_ref, z_ref, r_ref):
        xb = x_ref[...]
        z_ref[...] = _dot_t(xb, wl_ref[...])
        r_ref[...] = _dot_t(xb, wr_ref[...]) + b_ref[...]

    return pl.pallas_call(
        body,
        grid=(N // _BL,),
        in_specs=[
            pl.BlockSpec((_BL, D), lambda i: (i, 0)),
            pl.BlockSpec((D, D), lambda i: (0, 0)),
            pl.BlockSpec((D, D), lambda i: (0, 0)),
            pl.BlockSpec((1, D), lambda i: (0, 0)),
        ],
        out_specs=[pl.BlockSpec((_BL, D), lambda i: (i, 0)),
                   pl.BlockSpec((_BL, D), lambda i: (i, 0))],
        out_shape=[jax.ShapeDtypeStruct((N, D), jnp.float32)] * 2,
    )(x, Wl, Wr, b2d)


def _tc_mid(s_part, cnt_t, r_prev, Wl, Wr, b2d):
    _, N, D = s_part.shape

    def body(s_ref, c_ref, r_ref, wl_ref, wr_ref, b_ref, z_ref, r2_ref):
        cb = c_ref[...]
        rc = 1.0 / jnp.maximum(cb[:, 0] + cb[:, 1], 1.0)
        h = jnp.maximum((s_ref[0] + s_ref[1]) * rc[:, None] + r_ref[...], 0.0)
        z_ref[...] = _dot_t(h, wl_ref[...])
        r2_ref[...] = _dot_t(h, wr_ref[...]) + b_ref[...]

    return pl.pallas_call(
        body,
        grid=(N // _BL,),
        in_specs=[
            pl.BlockSpec((2, _BL, D), lambda i: (0, i, 0)),
            pl.BlockSpec((_BL, 2), lambda i: (i, 0)),
            pl.BlockSpec((_BL, D), lambda i: (i, 0)),
            pl.BlockSpec((D, D), lambda i: (0, 0)),
            pl.BlockSpec((D, D), lambda i: (0, 0)),
            pl.BlockSpec((1, D), lambda i: (0, 0)),
        ],
        out_specs=[pl.BlockSpec((_BL, D), lambda i: (i, 0)),
                   pl.BlockSpec((_BL, D), lambda i: (i, 0))],
        out_shape=[jax.ShapeDtypeStruct((N, D), jnp.float32)] * 2,
    )(s_part, cnt_t, r_prev, Wl, Wr, b2d)


def _tc_final(s_part, cnt_t, r_prev):
    _, N, D = s_part.shape

    def body(s_ref, c_ref, r_ref, o_ref):
        cb = c_ref[...]
        rc = 1.0 / jnp.maximum(cb[:, 0] + cb[:, 1], 1.0)
        h = (s_ref[0] + s_ref[1]) * rc[:, None] + r_ref[...]
        m = jnp.max(h, axis=1, keepdims=True)
        lse = jnp.log(jnp.sum(jnp.exp(h - m), axis=1, keepdims=True))
        o_ref[...] = h - m - lse

    return pl.pallas_call(
        body,
        grid=(N // _BL,),
        in_specs=[
            pl.BlockSpec((2, _BL, D), lambda i: (0, i, 0)),
            pl.BlockSpec((_BL, 2), lambda i: (i, 0)),
            pl.BlockSpec((_BL, D), lambda i: (i, 0)),
        ],
        out_specs=pl.BlockSpec((_BL, D), lambda i: (i, 0)),
        out_shape=jax.ShapeDtypeStruct((N, D), jnp.float32),
    )(s_part, cnt_t, r_prev)


def _sc_segment_sum(z, src2d, dst2d, zeros_rows, zeros_flat, ones_vec,
                    with_cnt: bool):
    N, D = z.shape
    EC = src2d.shape[0]
    q = EC // _NW
    assert EC % _NW == 0 and q % _QB == 0
    RS = (N // _NSUB) // 8 * 8
    RREM = N - RS * _NSUB
    ZREM = N + _NSUB - RS * _NSUB
    ZC = zeros_rows.shape[0]
    assert RS % ZC == 0 and ZC % 8 == 0 and ZREM <= ZC
    CPAD = zeros_flat.shape[0]
    NPAD = CPAD * _NSUB

    mesh = plsc.VectorSubcoreMesh(core_axis_name="c", subcore_axis_name="s")

    out_type = [jax.ShapeDtypeStruct((_NCORE, N, D), jnp.float32)]
    scratch = [
        pltpu.VMEM_SHARED((N + _NSUB, D), jnp.float32),
        pltpu.VMEM((_QB, _L), jnp.int32),
        pltpu.VMEM((_QB, _L), jnp.int32),
        pltpu.VMEM((_L, D), jnp.float32),
        pltpu.VMEM((_L, D), jnp.float32),
        pltpu.VMEM((ZC, D), jnp.float32),
        pltpu.SemaphoreType.DMA,
        pltpu.SemaphoreType.DMA,
    ]
    if with_cnt:
        out_type.append(jax.ShapeDtypeStruct((_NCORE * NPAD,), jnp.float32))
        scratch += [
            pltpu.VMEM_SHARED((NPAD,), jnp.float32),
            pltpu.VMEM((CPAD,), jnp.float32),
            pltpu.VMEM((_L,), jnp.float32),
        ]

    def body(z_h, src_h, dst_h, zr_h, zf_h, on_h, s_out, *rest):
        if with_cnt:
            (cnt_out, acc_sh, src_all, dst_all, rows0, rows1, stage_v,
             sem0, sem1, cnt_sh, cnt_stage, ones_v) = rest
        else:
            acc_sh, src_all, dst_all, rows0, rows1, stage_v, sem0, sem1 = rest
        c = lax.axis_index("c")
        s = lax.axis_index("s")
        w = s * _NCORE + c

        pltpu.sync_copy(zr_h, stage_v)
        for k in range(RS // ZC):
            pltpu.sync_copy(stage_v, acc_sh.at[pl.ds(s * RS + k * ZC, ZC), :])

        @pl.when(s == 0)
        def _():
            pltpu.sync_copy(stage_v.at[pl.ds(0, ZREM), :],
                            acc_sh.at[pl.ds(RS * _NSUB, ZREM), :])
        if with_cnt:
            pltpu.sync_copy(zf_h, cnt_stage)
            pltpu.sync_copy(cnt_stage, cnt_sh.at[pl.ds(s * CPAD, CPAD)])
            pltpu.sync_copy(on_h, ones_v)

        plsc.subcore_barrier()

        def block(b, carry):
            base = w * q + b * _QB
            pltpu.sync_copy(src_h.at[pl.ds(base, _QB), :], src_all)
            pltpu.sync_copy(dst_h.at[pl.ds(base, _QB), :], dst_all)

            for j in range(_QB):
                rows = rows0 if j % 2 == 0 else rows1
                pltpu.sync_copy(z_h.at[src_all.at[j]], rows)
                pltpu.sync_copy(rows, acc_sh.at[dst_all.at[j]], add=True)
                if with_cnt:
                    pltpu.sync_copy(ones_v, cnt_sh.at[dst_all.at[j]],
                                    add=True)
            return carry

        lax.fori_loop(0, q // _QB, block, 0)

        plsc.subcore_barrier()

        for k in range(RS // ZC):
            pltpu.sync_copy(acc_sh.at[pl.ds(s * RS + k * ZC, ZC), :], stage_v)
            pltpu.sync_copy(stage_v, s_out.at[c, pl.ds(s * RS + k * ZC, ZC), :])

        @pl.when(s == 0)
        def _():
            pltpu.sync_copy(acc_sh.at[pl.ds(RS * _NSUB, RREM), :],
                            stage_v.at[pl.ds(0, RREM), :])
            pltpu.sync_copy(stage_v.at[pl.ds(0, RREM), :],
                            s_out.at[c, pl.ds(RS * _NSUB, RREM), :])
        if with_cnt:
            pltpu.sync_copy(cnt_sh.at[pl.ds(s * CPAD, CPAD)], cnt_stage)
            pltpu.sync_copy(cnt_stage,
                            cnt_out.at[pl.ds(c * NPAD + s * CPAD, CPAD)])

    return pl.kernel(body, out_type=out_type, mesh=mesh,
                     scratch_types=scratch)(
        z, src2d, dst2d, zeros_rows, zeros_flat, ones_vec)


def kernel(x, edge_index, W1l, b1, W1r, W2l, b2, W2r, W3l, b3, W3r):
    N, D = x.shape
    E = edge_index.shape[1]
    assert N % _NSUB == 0

    EC = -(-E // _L)
    q = -(-EC // _NW)
    q = -(-q // _QB) * _QB
    EP = q * _NW * _L
    pad_src = jnp.arange(EP - E, dtype=jnp.int32) % N
    pad_dst = N + jnp.arange(EP - E, dtype=jnp.int32) % _NSUB
    src2d = jnp.concatenate([edge_index[0], pad_src]).reshape(-1, _L)
    dst2d = jnp.concatenate([edge_index[1], pad_dst]).reshape(-1, _L)
    base, extra = divmod(EC, _NW)
    perm = np.empty(q * _NW, np.int32)
    off = 0
    pad_off = EC
    for w_ in range(_NW):
        r = base + (1 if w_ < extra else 0)
        perm[w_ * q:w_ * q + r] = np.arange(off, off + r)
        perm[w_ * q + r:(w_ + 1) * q] = np.arange(pad_off, pad_off + q - r)
        off += r
        pad_off += q - r
    src2d = src2d[perm]
    dst2d = dst2d[perm]

    RS = (N // _NSUB) // 8 * 8
    ZC = next((c for c in (48, 24, 16, 8) if RS % c == 0), RS)
    CPAD = ((N // _NSUB + 7) // 8) * 8
    NPAD = CPAD * _NSUB
    zeros_rows = jnp.zeros((ZC, D), jnp.float32)
    zeros_flat = jnp.zeros((CPAD,), jnp.float32)
    ones_vec = jnp.ones((_L,), jnp.float32)

    b1d = b1.reshape(1, D)
    b2d = b2.reshape(1, D)
    b3d = b3.reshape(1, D)

    z1, r1 = _tc_entry(x, W1l, W1r, b1d)
    s1, cnt_pad = _sc_segment_sum(z1, src2d, dst2d, zeros_rows, zeros_flat,
                                  ones_vec, with_cnt=True)
    cnt_t = cnt_pad.reshape(_NCORE, NPAD)[:, :N].T

    z2, r2 = _tc_mid(s1, cnt_t, r1, W2l, W2r, b2d)
    s2 = _sc_segment_sum(z2, src2d, dst2d, zeros_rows, zeros_flat,
                         ones_vec, with_cnt=False)[0]

    z3, r3 = _tc_mid(s2, cnt_t, r2, W3l, W3r, b3d)
    s3 = _sc_segment_sum(z3, src2d, dst2d, zeros_rows, zeros_flat,
                         ones_vec, with_cnt=False)[0]

    return _tc_final(s3, cnt_t, r3)


# speedup vs baseline: 17.5048x; 1.3488x over previous
import functools

import jax
import jax.numpy as jnp
import numpy as np
from jax import lax
from jax.experimental import pallas as pl
from jax.experimental.pallas import tpu as pltpu
from jax.experimental.pallas import tpu_sc as plsc

_L = 128
_QB = 8
_NSUB = 16
_NCORE = 2
_NW = _NSUB * _NCORE
_BL = 1000


def _dot_t(a, w):
    return lax.dot_general(a, w, (((1,), (1,)), ((), ())),
                           preferred_element_type=jnp.float32)


def _tc_entry(x, Wl, Wr, b2d):
    N, D = x.shape

    def body(x_ref, wl_ref, wr_ref, b_ref, z_ref, r_ref):
        xb = x_ref[...]
        z_ref[...] = _dot_t(xb, wl_ref[...])
        r_ref[...] = _dot_t(xb, wr_ref[...]) + b_ref[...]

    return pl.pallas_call(
        body,
        grid=(N // _BL,),
        in_specs=[
            pl.BlockSpec((_BL, D), lambda i: (i, 0)),
            pl.BlockSpec((D, D), lambda i: (0, 0)),
            pl.BlockSpec((D, D), lambda i: (0, 0)),
            pl.BlockSpec((1, D), lambda i: (0, 0)),
        ],
        out_specs=[pl.BlockSpec((_BL, D), lambda i: (i, 0)),
                   pl.BlockSpec((_BL, D), lambda i: (i, 0))],
        out_shape=[jax.ShapeDtypeStruct((N, D), jnp.float32)] * 2,
    )(x, Wl, Wr, b2d)


def _tc_mid(s_part, cnt_t, r_prev, Wl, Wr, b2d):
    _, N, D = s_part.shape

    def body(s_ref, c_ref, r_ref, wl_ref, wr_ref, b_ref, z_ref, r2_ref):
        cb = c_ref[...]
        rc = 1.0 / jnp.maximum(cb[:, 0] + cb[:, 1], 1.0)
        h = jnp.maximum((s_ref[0] + s_ref[1]) * rc[:, None] + r_ref[...], 0.0)
        z_ref[...] = _dot_t(h, wl_ref[...])
        r2_ref[...] = _dot_t(h, wr_ref[...]) + b_ref[...]

    return pl.pallas_call(
        body,
        grid=(N // _BL,),
        in_specs=[
            pl.BlockSpec((2, _BL, D), lambda i: (0, i, 0)),
            pl.BlockSpec((_BL, 2), lambda i: (i, 0)),
            pl.BlockSpec((_BL, D), lambda i: (i, 0)),
            pl.BlockSpec((D, D), lambda i: (0, 0)),
            pl.BlockSpec((D, D), lambda i: (0, 0)),
            pl.BlockSpec((1, D), lambda i: (0, 0)),
        ],
        out_specs=[pl.BlockSpec((_BL, D), lambda i: (i, 0)),
                   pl.BlockSpec((_BL, D), lambda i: (i, 0))],
        out_shape=[jax.ShapeDtypeStruct((N, D), jnp.float32)] * 2,
    )(s_part, cnt_t, r_prev, Wl, Wr, b2d)


def _tc_final(s_part, cnt_t, r_prev):
    _, N, D = s_part.shape

    def body(s_ref, c_ref, r_ref, o_ref):
        cb = c_ref[...]
        rc = 1.0 / jnp.maximum(cb[:, 0] + cb[:, 1], 1.0)
        h = (s_ref[0] + s_ref[1]) * rc[:, None] + r_ref[...]
        m = jnp.max(h, axis=1, keepdims=True)
        lse = jnp.log(jnp.sum(jnp.exp(h - m), axis=1, keepdims=True))
        o_ref[...] = h - m - lse

    return pl.pallas_call(
        body,
        grid=(N // _BL,),
        in_specs=[
            pl.BlockSpec((2, _BL, D), lambda i: (0, i, 0)),
            pl.BlockSpec((_BL, 2), lambda i: (i, 0)),
            pl.BlockSpec((_BL, D), lambda i: (i, 0)),
        ],
        out_specs=pl.BlockSpec((_BL, D), lambda i: (i, 0)),
        out_shape=jax.ShapeDtypeStruct((N, D), jnp.float32),
    )(s_part, cnt_t, r_prev)


def _sc_segment_sum(z, src2d, dst2d, zeros_rows, zeros_flat, ones_vec,
                    with_cnt: bool):
    N, D = z.shape
    EC = src2d.shape[0]
    q = EC // _NW
    assert EC % _NW == 0 and q % _QB == 0
    RS = (N // _NSUB) // 8 * 8
    RREM = N - RS * _NSUB
    ZREM = N + _NSUB - RS * _NSUB
    ZC = zeros_rows.shape[0]
    assert RS % ZC == 0 and ZC % 8 == 0 and ZREM <= ZC
    CPAD = zeros_flat.shape[0]
    NPAD = CPAD * _NSUB

    mesh = plsc.VectorSubcoreMesh(core_axis_name="c", subcore_axis_name="s")

    out_type = [jax.ShapeDtypeStruct((_NCORE, N, D), jnp.float32)]
    scratch = [
        pltpu.VMEM_SHARED((N + _NSUB, D), jnp.float32),
        pltpu.VMEM((_QB, _L), jnp.int32),
        pltpu.VMEM((_QB, _L), jnp.int32),
        pltpu.VMEM((_L, D), jnp.float32),
        pltpu.VMEM((_L, D), jnp.float32),
        pltpu.VMEM((ZC, D), jnp.float32),
        pltpu.SemaphoreType.DMA,
        pltpu.SemaphoreType.DMA,
    ]
    if with_cnt:
        out_type.append(jax.ShapeDtypeStruct((_NCORE * NPAD,), jnp.float32))
        scratch += [
            pltpu.VMEM_SHARED((NPAD,), jnp.float32),
            pltpu.VMEM((CPAD,), jnp.float32),
            pltpu.VMEM((_L,), jnp.float32),
        ]

    def body(z_h, src_h, dst_h, zr_h, zf_h, on_h, s_out, *rest):
        if with_cnt:
            (cnt_out, acc_sh, src_all, dst_all, rows0, rows1, stage_v,
             sem0, sem1, cnt_sh, cnt_stage, ones_v) = rest
        else:
            acc_sh, src_all, dst_all, rows0, rows1, stage_v, sem0, sem1 = rest
        c = lax.axis_index("c")
        s = lax.axis_index("s")
        w = s * _NCORE + c

        pltpu.sync_copy(zr_h, stage_v)
        for k in range(RS // ZC):
            pltpu.sync_copy(stage_v, acc_sh.at[pl.ds(s * RS + k * ZC, ZC), :])

        @pl.when(s == 0)
        def _():
            pltpu.sync_copy(stage_v.at[pl.ds(0, ZREM), :],
                            acc_sh.at[pl.ds(RS * _NSUB, ZREM), :])
        if with_cnt:
            pltpu.sync_copy(zf_h, cnt_stage)
            pltpu.sync_copy(cnt_stage, cnt_sh.at[pl.ds(s * CPAD, CPAD)])
            pltpu.sync_copy(on_h, ones_v)

        plsc.subcore_barrier()

        def block(b, carry):
            base = w * q + b * _QB
            pltpu.sync_copy(src_h.at[pl.ds(base, _QB), :], src_all)
            pltpu.sync_copy(dst_h.at[pl.ds(base, _QB), :], dst_all)

            pltpu.async_copy(z_h.at[src_all.at[0]], rows0, sem0)
            for j in range(_QB):
                cur, csem = (rows0, sem0) if j % 2 == 0 else (rows1, sem1)
                if j + 1 < _QB:
                    nxt, nsem = (rows1, sem1) if j % 2 == 0 else (rows0, sem0)
                    pltpu.async_copy(z_h.at[src_all.at[j + 1]], nxt, nsem)
                pltpu.make_async_copy(z_h.at[src_all.at[j]], cur, csem).wait()
                pltpu.sync_copy(cur, acc_sh.at[dst_all.at[j]], add=True)
                if with_cnt:
                    pltpu.sync_copy(ones_v, cnt_sh.at[dst_all.at[j]],
                                    add=True)
            return carry

        lax.fori_loop(0, q // _QB, block, 0)

        plsc.subcore_barrier()

        for k in range(RS // ZC):
            pltpu.sync_copy(acc_sh.at[pl.ds(s * RS + k * ZC, ZC), :], stage_v)
            pltpu.sync_copy(stage_v, s_out.at[c, pl.ds(s * RS + k * ZC, ZC), :])

        @pl.when(s == 0)
        def _():
            pltpu.sync_copy(acc_sh.at[pl.ds(RS * _NSUB, RREM), :],
                            stage_v.at[pl.ds(0, RREM), :])
            pltpu.sync_copy(stage_v.at[pl.ds(0, RREM), :],
                            s_out.at[c, pl.ds(RS * _NSUB, RREM), :])
        if with_cnt:
            pltpu.sync_copy(cnt_sh.at[pl.ds(s * CPAD, CPAD)], cnt_stage)
            pltpu.sync_copy(cnt_stage,
                            cnt_out.at[pl.ds(c * NPAD + s * CPAD, CPAD)])

    return pl.kernel(body, out_type=out_type, mesh=mesh,
                     scratch_types=scratch)(
        z, src2d, dst2d, zeros_rows, zeros_flat, ones_vec)


def kernel(x, edge_index, W1l, b1, W1r, W2l, b2, W2r, W3l, b3, W3r):
    N, D = x.shape
    E = edge_index.shape[1]
    assert N % _NSUB == 0

    EC = -(-E // _L)
    q = -(-EC // _NW)
    q = -(-q // _QB) * _QB
    EP = q * _NW * _L
    pad_src = jnp.arange(EP - E, dtype=jnp.int32) % N
    pad_dst = N + jnp.arange(EP - E, dtype=jnp.int32) % _NSUB
    src2d = jnp.concatenate([edge_index[0], pad_src]).reshape(-1, _L)
    dst2d = jnp.concatenate([edge_index[1], pad_dst]).reshape(-1, _L)
    base, extra = divmod(EC, _NW)
    perm = np.empty(q * _NW, np.int32)
    off = 0
    pad_off = EC
    for w_ in range(_NW):
        r = base + (1 if w_ < extra else 0)
        perm[w_ * q:w_ * q + r] = np.arange(off, off + r)
        perm[w_ * q + r:(w_ + 1) * q] = np.arange(pad_off, pad_off + q - r)
        off += r
        pad_off += q - r
    src2d = src2d[perm]
    dst2d = dst2d[perm]

    RS = (N // _NSUB) // 8 * 8
    ZC = next((c for c in (48, 24, 16, 8) if RS % c == 0), RS)
    CPAD = ((N // _NSUB + 7) // 8) * 8
    NPAD = CPAD * _NSUB
    zeros_rows = jnp.zeros((ZC, D), jnp.float32)
    zeros_flat = jnp.zeros((CPAD,), jnp.float32)
    ones_vec = jnp.ones((_L,), jnp.float32)

    b1d = b1.reshape(1, D)
    b2d = b2.reshape(1, D)
    b3d = b3.reshape(1, D)

    z1, r1 = _tc_entry(x, W1l, W1r, b1d)
    s1, cnt_pad = _sc_segment_sum(z1, src2d, dst2d, zeros_rows, zeros_flat,
                                  ones_vec, with_cnt=True)
    cnt_t = cnt_pad.reshape(_NCORE, NPAD)[:, :N].T

    z2, r2 = _tc_mid(s1, cnt_t, r1, W2l, W2r, b2d)
    s2 = _sc_segment_sum(z2, src2d, dst2d, zeros_rows, zeros_flat,
                         ones_vec, with_cnt=False)[0]

    z3, r3 = _tc_mid(s2, cnt_t, r2, W3l, W3r, b3d)
    s3 = _sc_segment_sum(z3, src2d, dst2d, zeros_rows, zeros_flat,
                         ones_vec, with_cnt=False)[0]

    return _tc_final(s3, cnt_t, r3)


# speedup vs baseline: 17.5338x; 1.0017x over previous
import functools

import jax
import jax.numpy as jnp
import numpy as np
from jax import lax
from jax.experimental import pallas as pl
from jax.experimental.pallas import tpu as pltpu
from jax.experimental.pallas import tpu_sc as plsc

_L = 128
_QB = 8
_NSUB = 16
_NCORE = 2
_NW = _NSUB * _NCORE
_BL = 1000


def _dot_t(a, w):
    return lax.dot_general(a, w, (((1,), (1,)), ((), ())),
                           preferred_element_type=jnp.float32)


def _tc_entry(x, Wl, Wr, b2d):
    N, D = x.shape

    def body(x_ref, wl_ref, wr_ref, b_ref, z_ref, r_ref):
        xb = x_ref[...]
        z_ref[...] = _dot_t(xb, wl_ref[...])
        r_ref[...] = _dot_t(xb, wr_ref[...]) + b_ref[...]

    return pl.pallas_call(
        body,
        grid=(N // _BL,),
        in_specs=[
            pl.BlockSpec((_BL, D), lambda i: (i, 0)),
            pl.BlockSpec((D, D), lambda i: (0, 0)),
            pl.BlockSpec((D, D), lambda i: (0, 0)),
            pl.BlockSpec((1, D), lambda i: (0, 0)),
        ],
        out_specs=[pl.BlockSpec((_BL, D), lambda i: (i, 0)),
                   pl.BlockSpec((_BL, D), lambda i: (i, 0))],
        out_shape=[jax.ShapeDtypeStruct((N, D), jnp.float32)] * 2,
    )(x, Wl, Wr, b2d)


def _tc_mid(s_part, cnt_t, r_prev, Wl, Wr, b2d):
    _, N, D = s_part.shape

    def body(s_ref, c_ref, r_ref, wl_ref, wr_ref, b_ref, z_ref, r2_ref):
        cb = c_ref[...]
        rc = 1.0 / jnp.maximum(cb[:, 0] + cb[:, 1], 1.0)
        h = jnp.maximum((s_ref[0] + s_ref[1]) * rc[:, None] + r_ref[...], 0.0)
        z_ref[...] = _dot_t(h, wl_ref[...])
        r2_ref[...] = _dot_t(h, wr_ref[...]) + b_ref[...]

    return pl.pallas_call(
        body,
        grid=(N // _BL,),
        in_specs=[
            pl.BlockSpec((2, _BL, D), lambda i: (0, i, 0)),
            pl.BlockSpec((_BL, 2), lambda i: (i, 0)),
            pl.BlockSpec((_BL, D), lambda i: (i, 0)),
            pl.BlockSpec((D, D), lambda i: (0, 0)),
            pl.BlockSpec((D, D), lambda i: (0, 0)),
            pl.BlockSpec((1, D), lambda i: (0, 0)),
        ],
        out_specs=[pl.BlockSpec((_BL, D), lambda i: (i, 0)),
                   pl.BlockSpec((_BL, D), lambda i: (i, 0))],
        out_shape=[jax.ShapeDtypeStruct((N, D), jnp.float32)] * 2,
    )(s_part, cnt_t, r_prev, Wl, Wr, b2d)


def _tc_final(s_part, cnt_t, r_prev):
    _, N, D = s_part.shape

    def body(s_ref, c_ref, r_ref, o_ref):
        cb = c_ref[...]
        rc = 1.0 / jnp.maximum(cb[:, 0] + cb[:, 1], 1.0)
        h = (s_ref[0] + s_ref[1]) * rc[:, None] + r_ref[...]
        m = jnp.max(h, axis=1, keepdims=True)
        lse = jnp.log(jnp.sum(jnp.exp(h - m), axis=1, keepdims=True))
        o_ref[...] = h - m - lse

    return pl.pallas_call(
        body,
        grid=(N // _BL,),
        in_specs=[
            pl.BlockSpec((2, _BL, D), lambda i: (0, i, 0)),
            pl.BlockSpec((_BL, 2), lambda i: (i, 0)),
            pl.BlockSpec((_BL, D), lambda i: (i, 0)),
        ],
        out_specs=pl.BlockSpec((_BL, D), lambda i: (i, 0)),
        out_shape=jax.ShapeDtypeStruct((N, D), jnp.float32),
    )(s_part, cnt_t, r_prev)


def _sc_segment_sum(z, src2d, dst2d, zeros_rows, zeros_flat, ones_vec,
                    with_cnt: bool):
    N, D = z.shape
    EC = src2d.shape[0]
    q = EC // _NW
    assert EC % _NW == 0 and q % _QB == 0
    RS = (N // _NSUB) // 8 * 8
    RREM = N - RS * _NSUB
    ZREM = N + _NSUB - RS * _NSUB
    ZC = zeros_rows.shape[0]
    assert RS % ZC == 0 and ZC % 8 == 0 and ZREM <= ZC
    CPAD = zeros_flat.shape[0]
    NPAD = CPAD * _NSUB

    mesh = plsc.VectorSubcoreMesh(core_axis_name="c", subcore_axis_name="s")

    out_type = [jax.ShapeDtypeStruct((_NCORE, N, D), jnp.float32)]
    scratch = [
        pltpu.VMEM_SHARED((N + _NSUB, D), jnp.float32),
        pltpu.VMEM((_QB, _L), jnp.int32),
        pltpu.VMEM((_QB, _L), jnp.int32),
        pltpu.VMEM((_L, D), jnp.float32),
        pltpu.VMEM((_L, D), jnp.float32),
        pltpu.VMEM((ZC, D), jnp.float32),
        pltpu.SemaphoreType.DMA,
        pltpu.SemaphoreType.DMA,
        pltpu.SemaphoreType.DMA,
        pltpu.SemaphoreType.DMA,
    ]
    if with_cnt:
        out_type.append(jax.ShapeDtypeStruct((_NCORE * NPAD,), jnp.float32))
        scratch += [
            pltpu.VMEM_SHARED((NPAD,), jnp.float32),
            pltpu.VMEM((CPAD,), jnp.float32),
            pltpu.VMEM((_L,), jnp.float32),
        ]

    def body(z_h, src_h, dst_h, zr_h, zf_h, on_h, s_out, *rest):
        if with_cnt:
            (cnt_out, acc_sh, src_all, dst_all, rows0, rows1, stage_v,
             sem0, sem1, ssem0, ssem1, cnt_sh, cnt_stage, ones_v) = rest
        else:
            (acc_sh, src_all, dst_all, rows0, rows1, stage_v,
             sem0, sem1, ssem0, ssem1) = rest
        c = lax.axis_index("c")
        s = lax.axis_index("s")
        w = s * _NCORE + c

        pltpu.sync_copy(zr_h, stage_v)
        for k in range(RS // ZC):
            pltpu.sync_copy(stage_v, acc_sh.at[pl.ds(s * RS + k * ZC, ZC), :])

        @pl.when(s == 0)
        def _():
            pltpu.sync_copy(stage_v.at[pl.ds(0, ZREM), :],
                            acc_sh.at[pl.ds(RS * _NSUB, ZREM), :])
        if with_cnt:
            pltpu.sync_copy(zf_h, cnt_stage)
            pltpu.sync_copy(cnt_stage, cnt_sh.at[pl.ds(s * CPAD, CPAD)])
            pltpu.sync_copy(on_h, ones_v)

        plsc.subcore_barrier()

        def block(b, carry):
            base = w * q + b * _QB
            pltpu.sync_copy(src_h.at[pl.ds(base, _QB), :], src_all)
            pltpu.sync_copy(dst_h.at[pl.ds(base, _QB), :], dst_all)

            pltpu.async_copy(z_h.at[src_all.at[0]], rows0, sem0)
            for j in range(_QB):
                cur, csem, cssem = ((rows0, sem0, ssem0) if j % 2 == 0
                                    else (rows1, sem1, ssem1))
                if j + 1 < _QB:
                    nxt, nsem, nssem = ((rows1, sem1, ssem1) if j % 2 == 0
                                        else (rows0, sem0, ssem0))
                    if j - 1 >= 0:
                        pltpu.make_async_copy(
                            nxt, acc_sh.at[dst_all.at[j - 1]], nssem).wait()
                    pltpu.async_copy(z_h.at[src_all.at[j + 1]], nxt, nsem)
                pltpu.make_async_copy(z_h.at[src_all.at[j]], cur, csem).wait()
                pltpu.async_copy(cur, acc_sh.at[dst_all.at[j]], cssem,
                                 add=True)
                if with_cnt:
                    pltpu.sync_copy(ones_v, cnt_sh.at[dst_all.at[j]],
                                    add=True)
            pltpu.make_async_copy(rows0 if _QB % 2 == 1 else rows1,
                                  acc_sh.at[dst_all.at[_QB - 1]],
                                  ssem0 if _QB % 2 == 1 else ssem1).wait()
            pltpu.make_async_copy(rows1 if _QB % 2 == 1 else rows0,
                                  acc_sh.at[dst_all.at[_QB - 2]],
                                  ssem1 if _QB % 2 == 1 else ssem0).wait()
            return carry

        lax.fori_loop(0, q // _QB, block, 0)

        plsc.subcore_barrier()

        for k in range(RS // ZC):
            pltpu.sync_copy(acc_sh.at[pl.ds(s * RS + k * ZC, ZC), :], stage_v)
            pltpu.sync_copy(stage_v, s_out.at[c, pl.ds(s * RS + k * ZC, ZC), :])

        @pl.when(s == 0)
        def _():
            pltpu.sync_copy(acc_sh.at[pl.ds(RS * _NSUB, RREM), :],
                            stage_v.at[pl.ds(0, RREM), :])
            pltpu.sync_copy(stage_v.at[pl.ds(0, RREM), :],
                            s_out.at[c, pl.ds(RS * _NSUB, RREM), :])
        if with_cnt:
            pltpu.sync_copy(cnt_sh.at[pl.ds(s * CPAD, CPAD)], cnt_stage)
            pltpu.sync_copy(cnt_stage,
                            cnt_out.at[pl.ds(c * NPAD + s * CPAD, CPAD)])

    return pl.kernel(body, out_type=out_type, mesh=mesh,
                     scratch_types=scratch)(
        z, src2d, dst2d, zeros_rows, zeros_flat, ones_vec)


def kernel(x, edge_index, W1l, b1, W1r, W2l, b2, W2r, W3l, b3, W3r):
    N, D = x.shape
    E = edge_index.shape[1]
    assert N % _NSUB == 0

    EC = -(-E // _L)
    q = -(-EC // _NW)
    q = -(-q // _QB) * _QB
    EP = q * _NW * _L
    pad_src = jnp.arange(EP - E, dtype=jnp.int32) % N
    pad_dst = N + jnp.arange(EP - E, dtype=jnp.int32) % _NSUB
    src2d = jnp.concatenate([edge_index[0], pad_src]).reshape(-1, _L)
    dst2d = jnp.concatenate([edge_index[1], pad_dst]).reshape(-1, _L)
    base, extra = divmod(EC, _NW)
    perm = np.empty(q * _NW, np.int32)
    off = 0
    pad_off = EC
    for w_ in range(_NW):
        r = base + (1 if w_ < extra else 0)
        perm[w_ * q:w_ * q + r] = np.arange(off, off + r)
        perm[w_ * q + r:(w_ + 1) * q] = np.arange(pad_off, pad_off + q - r)
        off += r
        pad_off += q - r
    src2d = src2d[perm]
    dst2d = dst2d[perm]

    RS = (N // _NSUB) // 8 * 8
    ZC = next((c for c in (48, 24, 16, 8) if RS % c == 0), RS)
    CPAD = ((N // _NSUB + 7) // 8) * 8
    NPAD = CPAD * _NSUB
    zeros_rows = jnp.zeros((ZC, D), jnp.float32)
    zeros_flat = jnp.zeros((CPAD,), jnp.float32)
    ones_vec = jnp.ones((_L,), jnp.float32)

    b1d = b1.reshape(1, D)
    b2d = b2.reshape(1, D)
    b3d = b3.reshape(1, D)

    z1, r1 = _tc_entry(x, W1l, W1r, b1d)
    s1, cnt_pad = _sc_segment_sum(z1, src2d, dst2d, zeros_rows, zeros_flat,
                                  ones_vec, with_cnt=True)
    cnt_t = cnt_pad.reshape(_NCORE, NPAD)[:, :N].T

    z2, r2 = _tc_mid(s1, cnt_t, r1, W2l, W2r, b2d)
    s2 = _sc_segment_sum(z2, src2d, dst2d, zeros_rows, zeros_flat,
                         ones_vec, with_cnt=False)[0]

    z3, r3 = _tc_mid(s2, cnt_t, r2, W3l, W3r, b3d)
    s3 = _sc_segment_sum(z3, src2d, dst2d, zeros_rows, zeros_flat,
                         ones_vec, with_cnt=False)[0]

    return _tc_final(s3, cnt_t, r3)
